# Initial kernel scaffold; baseline (speedup 1.0000x reference)
import jax
import jax.numpy as jnp
from jax.experimental import pallas as pl


def kernel(tokens, router_w, router_b, w1, b1, w2, b2):
    raise NotImplementedError("write your pallas kernel here")


# speedup vs baseline: 26.7764x; 26.7764x over previous
import jax
import jax.numpy as jnp
from jax.experimental import pallas as pl
from jax.experimental.pallas import tpu as pltpu

_HIDDEN = 768
_NUM_EXPERTS = 8
_EXPANDED = 1536
_N_TOKENS = 128


def _moe_body(tokens_ref, router_w_ref, router_b_ref, w1_ref, b1_ref,
              w2_ref, b2_ref, out_ref, p_scratch):
    e = pl.program_id(0)
    tokens = tokens_ref[...]

    @pl.when(e == 0)
    def _():
        logits = jnp.dot(tokens, router_w_ref[...],
                         preferred_element_type=jnp.float32) + router_b_ref[...]
        col = jax.lax.broadcasted_iota(jnp.int32, logits.shape, 1)
        m1 = jnp.max(logits, axis=-1, keepdims=True)
        idx1 = jnp.min(jnp.where(logits == m1, col, _NUM_EXPERTS),
                       axis=-1, keepdims=True)
        oh1 = col == idx1
        masked = jnp.where(oh1, -jnp.inf, logits)
        m2 = jnp.max(masked, axis=-1, keepdims=True)
        idx2 = jnp.min(jnp.where(masked == m2, col, _NUM_EXPERTS),
                       axis=-1, keepdims=True)
        oh2 = col == idx2
        p1 = 1.0 / (1.0 + jnp.exp(m2 - m1))
        p_scratch[...] = jnp.where(oh1, p1, 0.0) + jnp.where(oh2, 1.0 - p1, 0.0)
        out_ref[...] = jnp.zeros_like(out_ref)

    h = jnp.dot(tokens, w1_ref[0], preferred_element_type=jnp.float32) + b1_ref[0, 0]
    h = h * 0.5 * (1.0 + jax.lax.erf(h * 0.7071067811865476))
    o = jnp.dot(h, w2_ref[0], preferred_element_type=jnp.float32) + b2_ref[0, 0]
    col = jax.lax.broadcasted_iota(jnp.int32, p_scratch.shape, 1)
    p_col = jnp.sum(jnp.where(col == e, p_scratch[...], 0.0),
                    axis=-1, keepdims=True)
    out_ref[...] += p_col * o


def kernel(tokens, router_w, router_b, w1, b1, w2, b2):
    return pl.pallas_call(
        _moe_body,
        grid=(_NUM_EXPERTS,),
        in_specs=[
            pl.BlockSpec((_N_TOKENS, _HIDDEN), lambda e: (0, 0)),
            pl.BlockSpec((_HIDDEN, _NUM_EXPERTS), lambda e: (0, 0)),
            pl.BlockSpec((1, _NUM_EXPERTS), lambda e: (0, 0)),
            pl.BlockSpec((1, _HIDDEN, _EXPANDED), lambda e: (e, 0, 0)),
            pl.BlockSpec((1, 1, _EXPANDED), lambda e: (e, 0, 0)),
            pl.BlockSpec((1, _EXPANDED, _HIDDEN), lambda e: (e, 0, 0)),
            pl.BlockSpec((1, 1, _HIDDEN), lambda e: (e, 0, 0)),
        ],
        out_specs=pl.BlockSpec((_N_TOKENS, _HIDDEN), lambda e: (0, 0)),
        out_shape=jax.ShapeDtypeStruct((_N_TOKENS, _HIDDEN), tokens.dtype),
        scratch_shapes=[pltpu.VMEM((_N_TOKENS, _NUM_EXPERTS), jnp.float32)],
    )(tokens, router_w, router_b.reshape(1, -1), w1,
      b1.reshape(_NUM_EXPERTS, 1, _EXPANDED), w2,
      b2.reshape(_NUM_EXPERTS, 1, _HIDDEN))
